# baseline (speedup 1.0000x reference)
import functools

import jax
import jax.numpy as jnp
from jax import lax
from jax.experimental import pallas as pl
from jax.experimental.pallas import tpu as pltpu
from jax.experimental.pallas import tpu_sc as plsc

N = 10000
P = 4
E = 160000
D = 256
H = 64
HID = 128
OUT = 3
ALPHA = 0.15

NC = 2
NS = 16
EW = E // NS
CW = 128
EWP = 10112
NCH = EWP // CW
RC = 80
NRC = N // RC
MAXJ = (NRC + NS - 1) // NS
NB = 10
BN = N // NB


def _fc_body(x_ref, w_ref, b_ref, s_ref, h0_ref, al_ref):
    h = jnp.dot(x_ref[...], w_ref[...], preferred_element_type=jnp.float32)
    h = jnp.maximum(h + b_ref[...], 0.0)
    h0_ref[...] = h
    al_ref[...] = ALPHA * s_ref[...] * h


_fc = pl.pallas_call(
    _fc_body,
    grid=(NB,),
    in_specs=[
        pl.BlockSpec((BN, D), lambda i: (i, 0)),
        pl.BlockSpec((D, H), lambda i: (0, 0)),
        pl.BlockSpec((1, H), lambda i: (0, 0)),
        pl.BlockSpec((BN, 1), lambda i: (i, 0)),
    ],
    out_specs=[
        pl.BlockSpec((BN, H), lambda i: (i, 0)),
        pl.BlockSpec((BN, H), lambda i: (i, 0)),
    ],
    out_shape=[
        jax.ShapeDtypeStruct((N, H), jnp.float32),
        jax.ShapeDtypeStruct((N + RC, H), jnp.float32),
    ],
)


NBUF = 3
NGRP = NCH // NBUF
NTAIL = NCH - NGRP * NBUF


def _edge_pass(table_ref, acc_g, deg_g, src_v, dst_v, msg_v, ones_v,
               gsem, ssem, dsem, *, with_deg):
    for t in range(NBUF):
        pltpu.async_copy(table_ref.at[src_v.at[t]], msg_v.at[t], gsem.at[t])

    def _grp(j2, _):
        j0 = j2 * NBUF
        for t in range(NBUF):
            j = j0 + t
            pltpu.make_async_copy(
                table_ref.at[src_v.at[t]], msg_v.at[t], gsem.at[t]).wait()
            if with_deg:
                pltpu.async_copy(ones_v, deg_g.at[dst_v.at[j]], dsem, add=True)
            pltpu.async_copy(msg_v.at[t], acc_g.at[dst_v.at[j]], ssem.at[t],
                             add=True)
        for t in range(NBUF):
            j = j0 + t
            pltpu.make_async_copy(
                msg_v.at[t], acc_g.at[dst_v.at[j]], ssem.at[t]).wait()
            nxt = jnp.minimum(j + NBUF, NCH - 1)
            pltpu.async_copy(table_ref.at[src_v.at[nxt]], msg_v.at[t],
                             gsem.at[t])
        return _
    lax.fori_loop(0, NGRP, _grp, None)

    assert NTAIL == 1
    pltpu.make_async_copy(
        table_ref.at[src_v.at[0]], msg_v.at[0], gsem.at[0]).wait()
    if with_deg:
        pltpu.async_copy(ones_v, deg_g.at[dst_v.at[NCH - 1]], dsem, add=True)
    pltpu.sync_copy(msg_v.at[0], acc_g.at[dst_v.at[NCH - 1]], add=True)
    for t in range(1, NBUF):
        pltpu.make_async_copy(
            table_ref.at[src_v.at[0]], msg_v.at[t], gsem.at[t]).wait()
    if with_deg:
        def _dwait(j, _):
            pltpu.make_async_copy(ones_v, deg_g.at[dst_v.at[0]], dsem).wait()
            return _
        lax.fori_loop(0, NCH, _dwait, None)


def _combine(p, s, z_ref, alpha_ref, z64_ref, z16_ref, acc_g, deg_g,
             deg_c, acc_c, al_c, out_c, csem, *, first):
    def _base(k):
        return jnp.minimum(s + k * NS, NRC) * RC

    def _deg_cp(k, b):
        return pltpu.make_async_copy(
            deg_g.at[pl.ds(_base(k), RC), :], deg_c.at[b], csem.at[0, b])

    def _acc_cp(k, b):
        return pltpu.make_async_copy(
            acc_g.at[pl.ds(_base(k), RC), :], acc_c.at[b], csem.at[1, b])

    def _al_cp(k, b):
        return pltpu.make_async_copy(
            alpha_ref.at[pl.ds(_base(k), RC), :], al_c.at[b], csem.at[2, b])

    def _zst_cp(k, b):
        return pltpu.make_async_copy(
            out_c.at[b], z_ref.at[p, pl.ds(_base(k), RC), :], csem.at[3, b])

    def _rz_cp(k, b):
        return pltpu.make_async_copy(
            z64_ref, acc_g.at[pl.ds(_base(k), RC), :], csem.at[4, b])

    def _rz16_cp(k, b):
        return pltpu.make_async_copy(
            z16_ref, deg_g.at[pl.ds(_base(k), RC), :], csem.at[5, b])

    def _issue_loads(k, b):
        _deg_cp(k, b).start()
        _acc_cp(k, b).start()
        _al_cp(k, b).start()

    def _wait_loads(k, b):
        _deg_cp(k, b).wait()
        _acc_cp(k, b).wait()
        _al_cp(k, b).wait()

    def _issue_stores(k, b):
        _zst_cp(k, b).start()
        _rz_cp(k, b).start()
        if not first:
            _rz16_cp(k, b).start()

    def _wait_stores(k, b):
        _zst_cp(k, b).wait()
        _rz_cp(k, b).wait()
        if not first:
            _rz16_cp(k, b).wait()

    _issue_loads(0, 0)
    for k in range(MAXJ):
        b = k % 2
        if k + 1 < MAXJ:
            _issue_loads(k + 1, 1 - b)
        if k >= 2:
            _wait_stores(k - 2, b)
        _wait_loads(k, b)

        def _rows(r, _2, b=b):
            rd = (1.0 - ALPHA) / jnp.maximum(deg_c[b, r, :], 1.0)
            for u in range(4):
                sl = pl.ds(u * 16, 16)
                out_c[b, r, sl] = acc_c[b, r, sl] * rd + al_c[b, r, sl]
            return _2
        lax.fori_loop(0, RC, _rows, None)
        _issue_stores(k, b)
    for k in (MAXJ - 2, MAXJ - 1):
        _wait_stores(k, k % 2)


def _sc_body(edge_ref, h0_ref, alpha_ref, ones_ref, z64_ref, z16_ref, z_ref,
             acc_sh, deg_sh, src_v, dst_v, msg_v, ones_v,
             deg_c, acc_c, al_c, out_c, gsem, ssem, dsem, csem):
    c = lax.axis_index("c")
    s = lax.axis_index("s")

    pltpu.sync_copy(ones_ref, ones_v)

    def _zero(k, _):
        base = jnp.minimum(s + k * NS, NRC) * RC
        pltpu.sync_copy(z64_ref, acc_sh.at[pl.ds(base, RC), :])
        pltpu.sync_copy(z16_ref, deg_sh.at[pl.ds(base, RC), :])
        return _
    lax.fori_loop(0, MAXJ, _zero, None)

    for g in range(2):
        p = 2 * c + g
        acc_g = acc_sh
        deg_g = deg_sh

        pltpu.sync_copy(edge_ref.at[c, g, 0, s], src_v)
        pltpu.sync_copy(edge_ref.at[c, g, 1, s], dst_v)
        plsc.subcore_barrier()

        _edge_pass(h0_ref, acc_g, deg_g, src_v, dst_v, msg_v, ones_v,
                   gsem, ssem, dsem, with_deg=True)
        plsc.subcore_barrier()

        _combine(p, s, z_ref, alpha_ref, z64_ref, z16_ref, acc_g, deg_g,
                 deg_c, acc_c, al_c, out_c, csem, first=True)
        plsc.subcore_barrier()

        _edge_pass(z_ref.at[p], acc_g, deg_g, src_v, dst_v, msg_v, ones_v,
                   gsem, ssem, dsem, with_deg=False)
        plsc.subcore_barrier()

        _combine(p, s, z_ref, alpha_ref, z64_ref, z16_ref, acc_g, deg_g,
                 deg_c, acc_c, al_c, out_c, csem, first=False)


_sc_call = pl.kernel(
    _sc_body,
    out_type=jax.ShapeDtypeStruct((P, N + RC, H), jnp.float32),
    mesh=plsc.VectorSubcoreMesh(
        core_axis_name="c", subcore_axis_name="s",
        num_cores=NC, num_subcores=NS),
    compiler_params=pltpu.CompilerParams(use_tc_tiling_on_sc=False),
    scratch_types=[
        pltpu.VMEM_SHARED((N + RC, H), jnp.float32),
        pltpu.VMEM_SHARED((N + RC, 16), jnp.float32),
        pltpu.VMEM((NCH, CW), jnp.int32),
        pltpu.VMEM((NCH, CW), jnp.int32),
        pltpu.VMEM((NBUF, CW, H), jnp.float32),
        pltpu.VMEM((CW, 16), jnp.float32),
        pltpu.VMEM((2, RC, 16), jnp.float32),
        pltpu.VMEM((2, RC, H), jnp.float32),
        pltpu.VMEM((2, RC, H), jnp.float32),
        pltpu.VMEM((2, RC, H), jnp.float32),
        pltpu.SemaphoreType.DMA((NBUF,)),
        pltpu.SemaphoreType.DMA((NBUF,)),
        pltpu.SemaphoreType.DMA,
        pltpu.SemaphoreType.DMA((6, 2)),
    ],
)


def _att_body(z_ref, w1_ref, b1_ref, w2_ref, wp_ref, bp_ref, a_ref, h_ref, sg):
    ph = pl.program_id(0)
    i = pl.program_id(1)

    @pl.when(ph == 0)
    def _phase0():
        @pl.when(i == 0)
        def _init():
            sg[...] = jnp.zeros((P, HID), jnp.float32)
        zb = z_ref[...]
        parts = []
        for q in range(P):
            tq = jnp.dot(zb[q], w1_ref[...], preferred_element_type=jnp.float32)
            tq = jnp.tanh(tq + b1_ref[...])
            parts.append(jnp.sum(tq, axis=0))
        sg[...] = sg[...] + jnp.stack(parts)

    @pl.when(ph == 1)
    def _phase1():
        S = sg[...]
        w = jnp.dot(S, w2_ref[...], preferred_element_type=jnp.float32) / N
        w = w - jnp.max(w, axis=0, keepdims=True)
        ew = jnp.exp(w)
        beta = ew / jnp.sum(ew, axis=0, keepdims=True)
        zb = z_ref[...]
        hacc = beta[0, 0] * zb[0]
        for q in range(1, P):
            hacc = hacc + beta[q, 0] * zb[q]
        h_ref[...] = hacc
        a_ref[...] = jnp.dot(
            hacc, wp_ref[...], preferred_element_type=jnp.float32) + bp_ref[...]


_att = pl.pallas_call(
    _att_body,
    grid=(2, NB),
    in_specs=[
        pl.BlockSpec((P, BN, H), lambda ph, i: (0, i, 0)),
        pl.BlockSpec((H, HID), lambda ph, i: (0, 0)),
        pl.BlockSpec((1, HID), lambda ph, i: (0, 0)),
        pl.BlockSpec((HID, 1), lambda ph, i: (0, 0)),
        pl.BlockSpec((H, OUT), lambda ph, i: (0, 0)),
        pl.BlockSpec((1, OUT), lambda ph, i: (0, 0)),
    ],
    out_specs=[
        pl.BlockSpec((BN, OUT), lambda ph, i: (i, 0)),
        pl.BlockSpec((BN, H), lambda ph, i: (i, 0)),
    ],
    out_shape=[
        jax.ShapeDtypeStruct((N, OUT), jnp.float32),
        jax.ShapeDtypeStruct((N, H), jnp.float32),
    ],
    scratch_shapes=[pltpu.VMEM((P, HID), jnp.float32)],
)


def kernel(features, edge_index, simlar, W_fc, b_fc, W1, b1, W2, Wp, bp):
    x = features[0]
    h0, alpha_t = _fc(x, W_fc, b_fc.reshape(1, H), simlar.reshape(N, 1))
    ei = edge_index.reshape(P, 2, NS, EW)
    srcp = jnp.pad(ei[:, 0], ((0, 0), (0, 0), (0, EWP - EW)),
                   constant_values=0)
    dstp = jnp.pad(ei[:, 1], ((0, 0), (0, 0), (0, EWP - EW)),
                   constant_values=N)
    edge_r = jnp.stack([srcp, dstp], axis=1).reshape(NC, 2, 2, NS, NCH, CW)
    ones = jnp.ones((CW, 16), jnp.float32)
    z64 = jnp.zeros((RC, H), jnp.float32)
    z16 = jnp.zeros((RC, 16), jnp.float32)
    z = _sc_call(edge_r, h0, alpha_t, ones, z64, z16)
    a, h = _att(z, W1, b1.reshape(1, HID), W2, Wp, bp.reshape(1, OUT))
    return (a, h)


# speedup vs baseline: 2.3974x; 2.3974x over previous
import functools

import jax
import jax.numpy as jnp
from jax import lax
from jax.experimental import pallas as pl
from jax.experimental.pallas import tpu as pltpu
from jax.experimental.pallas import tpu_sc as plsc

N = 10000
P = 4
E = 160000
D = 256
H = 64
HID = 128
OUT = 3
ALPHA = 0.15

NC = 2
NS = 16
EW = E // NS
CW = 80
NCH = EW // CW
RC = 80
NRC = N // RC
MAXJ = (NRC + NS - 1) // NS
NB = 10
BN = N // NB


def _fc_body(x_ref, w_ref, b_ref, s_ref, h0_ref, al_ref):
    h = jnp.dot(x_ref[...], w_ref[...], preferred_element_type=jnp.float32)
    h = jnp.maximum(h + b_ref[...], 0.0)
    h0_ref[...] = h
    al_ref[...] = ALPHA * s_ref[...] * h


_fc = pl.pallas_call(
    _fc_body,
    grid=(NB,),
    in_specs=[
        pl.BlockSpec((BN, D), lambda i: (i, 0)),
        pl.BlockSpec((D, H), lambda i: (0, 0)),
        pl.BlockSpec((1, H), lambda i: (0, 0)),
        pl.BlockSpec((BN, 1), lambda i: (i, 0)),
    ],
    out_specs=[
        pl.BlockSpec((BN, H), lambda i: (i, 0)),
        pl.BlockSpec((BN, H), lambda i: (i, 0)),
    ],
    out_shape=[
        jax.ShapeDtypeStruct((N, H), jnp.float32),
        jax.ShapeDtypeStruct((N + RC, H), jnp.float32),
    ],
)


NBUF = 5
NGRP = NCH // NBUF


def _edge_pass(table_ref, acc_g, deg_g, src_v, dst_v, msg_v, ones_v,
               gsem, ssem, dsem, *, with_deg):
    for t in range(NBUF):
        pltpu.async_copy(table_ref.at[src_v.at[t]], msg_v.at[t], gsem.at[t])

    def _grp(j2, _):
        j0 = j2 * NBUF
        for t in range(NBUF):
            j = j0 + t
            pltpu.make_async_copy(
                table_ref.at[src_v.at[t]], msg_v.at[t], gsem.at[t]).wait()
            if with_deg:
                pltpu.async_copy(ones_v, deg_g.at[dst_v.at[j]], dsem, add=True)
            pltpu.async_copy(msg_v.at[t], acc_g.at[dst_v.at[j]], ssem.at[t],
                             add=True)
        for t in range(NBUF):
            j = j0 + t
            pltpu.make_async_copy(
                msg_v.at[t], acc_g.at[dst_v.at[j]], ssem.at[t]).wait()
            nxt = jnp.minimum(j + NBUF, NCH - 1)
            pltpu.async_copy(table_ref.at[src_v.at[nxt]], msg_v.at[t],
                             gsem.at[t])
        return _
    lax.fori_loop(0, NGRP, _grp, None)

    for t in range(NBUF):
        pltpu.make_async_copy(
            table_ref.at[src_v.at[0]], msg_v.at[t], gsem.at[t]).wait()
    if with_deg:
        def _dwait(j, _):
            pltpu.make_async_copy(ones_v, deg_g.at[dst_v.at[0]], dsem).wait()
            return _
        lax.fori_loop(0, NCH, _dwait, None)


def _combine(p, s, z_ref, alpha_ref, z64_ref, z16_ref, acc_g, deg_g,
             deg_c, acc_c, al_c, out_c, csem, *, first):
    def _base(k):
        return jnp.minimum(s + k * NS, NRC) * RC

    def _deg_cp(k, b):
        return pltpu.make_async_copy(
            deg_g.at[pl.ds(_base(k), RC), :], deg_c.at[b], csem.at[0, b])

    def _acc_cp(k, b):
        return pltpu.make_async_copy(
            acc_g.at[pl.ds(_base(k), RC), :], acc_c.at[b], csem.at[1, b])

    def _al_cp(k, b):
        return pltpu.make_async_copy(
            alpha_ref.at[pl.ds(_base(k), RC), :], al_c.at[b], csem.at[2, b])

    def _zst_cp(k, b):
        return pltpu.make_async_copy(
            out_c.at[b], z_ref.at[p, pl.ds(_base(k), RC), :], csem.at[3, b])

    def _rz_cp(k, b):
        return pltpu.make_async_copy(
            z64_ref, acc_g.at[pl.ds(_base(k), RC), :], csem.at[4, b])

    def _rz16_cp(k, b):
        return pltpu.make_async_copy(
            z16_ref, deg_g.at[pl.ds(_base(k), RC), :], csem.at[5, b])

    def _issue_loads(k, b):
        _deg_cp(k, b).start()
        _acc_cp(k, b).start()
        _al_cp(k, b).start()

    def _wait_loads(k, b):
        _deg_cp(k, b).wait()
        _acc_cp(k, b).wait()
        _al_cp(k, b).wait()

    def _issue_stores(k, b):
        _zst_cp(k, b).start()
        _rz_cp(k, b).start()
        if not first:
            _rz16_cp(k, b).start()

    def _wait_stores(k, b):
        _zst_cp(k, b).wait()
        _rz_cp(k, b).wait()
        if not first:
            _rz16_cp(k, b).wait()

    _issue_loads(0, 0)
    for k in range(MAXJ):
        b = k % 2
        if k + 1 < MAXJ:
            _issue_loads(k + 1, 1 - b)
        if k >= 2:
            _wait_stores(k - 2, b)
        _wait_loads(k, b)

        def _rows(r, _2, b=b):
            rd = (1.0 - ALPHA) / jnp.maximum(deg_c[b, r, :], 1.0)
            for u in range(4):
                sl = pl.ds(u * 16, 16)
                out_c[b, r, sl] = acc_c[b, r, sl] * rd + al_c[b, r, sl]
            return _2
        lax.fori_loop(0, RC, _rows, None)
        _issue_stores(k, b)
    for k in (MAXJ - 2, MAXJ - 1):
        _wait_stores(k, k % 2)


def _sc_body(edge_ref, h0_ref, alpha_ref, ones_ref, z64_ref, z16_ref, z_ref,
             acc_sh, deg_sh, src_v, dst_v, msg_v, ones_v,
             deg_c, acc_c, al_c, out_c, gsem, ssem, dsem, csem):
    c = lax.axis_index("c")
    s = lax.axis_index("s")

    pltpu.sync_copy(ones_ref, ones_v)

    def _zero(k, _):
        base = jnp.minimum(s + k * NS, NRC) * RC
        pltpu.sync_copy(z64_ref, acc_sh.at[pl.ds(base, RC), :])
        pltpu.sync_copy(z16_ref, deg_sh.at[pl.ds(base, RC), :])
        return _
    lax.fori_loop(0, MAXJ, _zero, None)

    for g in range(2):
        p = 2 * c + g
        acc_g = acc_sh
        deg_g = deg_sh

        pltpu.sync_copy(edge_ref.at[c, g, 0, s], src_v)
        pltpu.sync_copy(edge_ref.at[c, g, 1, s], dst_v)
        plsc.subcore_barrier()

        _edge_pass(h0_ref, acc_g, deg_g, src_v, dst_v, msg_v, ones_v,
                   gsem, ssem, dsem, with_deg=True)
        plsc.subcore_barrier()

        _combine(p, s, z_ref, alpha_ref, z64_ref, z16_ref, acc_g, deg_g,
                 deg_c, acc_c, al_c, out_c, csem, first=True)
        plsc.subcore_barrier()

        _edge_pass(z_ref.at[p], acc_g, deg_g, src_v, dst_v, msg_v, ones_v,
                   gsem, ssem, dsem, with_deg=False)
        plsc.subcore_barrier()

        _combine(p, s, z_ref, alpha_ref, z64_ref, z16_ref, acc_g, deg_g,
                 deg_c, acc_c, al_c, out_c, csem, first=False)


_sc_call = pl.kernel(
    _sc_body,
    out_type=jax.ShapeDtypeStruct((P, N + RC, H), jnp.float32),
    mesh=plsc.VectorSubcoreMesh(
        core_axis_name="c", subcore_axis_name="s",
        num_cores=NC, num_subcores=NS),
    compiler_params=pltpu.CompilerParams(use_tc_tiling_on_sc=False),
    scratch_types=[
        pltpu.VMEM_SHARED((N + RC, H), jnp.float32),
        pltpu.VMEM_SHARED((N + RC, 16), jnp.float32),
        pltpu.VMEM((NCH, CW), jnp.int32),
        pltpu.VMEM((NCH, CW), jnp.int32),
        pltpu.VMEM((NBUF, CW, H), jnp.float32),
        pltpu.VMEM((CW, 16), jnp.float32),
        pltpu.VMEM((2, RC, 16), jnp.float32),
        pltpu.VMEM((2, RC, H), jnp.float32),
        pltpu.VMEM((2, RC, H), jnp.float32),
        pltpu.VMEM((2, RC, H), jnp.float32),
        pltpu.SemaphoreType.DMA((NBUF,)),
        pltpu.SemaphoreType.DMA((NBUF,)),
        pltpu.SemaphoreType.DMA,
        pltpu.SemaphoreType.DMA((6, 2)),
    ],
)


def _att_body(z_ref, w1_ref, b1_ref, w2_ref, wp_ref, bp_ref, a_ref, h_ref, sg):
    ph = pl.program_id(0)
    i = pl.program_id(1)

    @pl.when(ph == 0)
    def _phase0():
        @pl.when(i == 0)
        def _init():
            sg[...] = jnp.zeros((P, HID), jnp.float32)
        zb = z_ref[...]
        parts = []
        for q in range(P):
            tq = jnp.dot(zb[q], w1_ref[...], preferred_element_type=jnp.float32)
            tq = jnp.tanh(tq + b1_ref[...])
            parts.append(jnp.sum(tq, axis=0))
        sg[...] = sg[...] + jnp.stack(parts)

    @pl.when(ph == 1)
    def _phase1():
        S = sg[...]
        w = jnp.dot(S, w2_ref[...], preferred_element_type=jnp.float32) / N
        w = w - jnp.max(w, axis=0, keepdims=True)
        ew = jnp.exp(w)
        beta = ew / jnp.sum(ew, axis=0, keepdims=True)
        zb = z_ref[...]
        hacc = beta[0, 0] * zb[0]
        for q in range(1, P):
            hacc = hacc + beta[q, 0] * zb[q]
        h_ref[...] = hacc
        a_ref[...] = jnp.dot(
            hacc, wp_ref[...], preferred_element_type=jnp.float32) + bp_ref[...]


_att = pl.pallas_call(
    _att_body,
    grid=(2, NB),
    in_specs=[
        pl.BlockSpec((P, BN, H), lambda ph, i: (0, i, 0)),
        pl.BlockSpec((H, HID), lambda ph, i: (0, 0)),
        pl.BlockSpec((1, HID), lambda ph, i: (0, 0)),
        pl.BlockSpec((HID, 1), lambda ph, i: (0, 0)),
        pl.BlockSpec((H, OUT), lambda ph, i: (0, 0)),
        pl.BlockSpec((1, OUT), lambda ph, i: (0, 0)),
    ],
    out_specs=[
        pl.BlockSpec((BN, OUT), lambda ph, i: (i, 0)),
        pl.BlockSpec((BN, H), lambda ph, i: (i, 0)),
    ],
    out_shape=[
        jax.ShapeDtypeStruct((N, OUT), jnp.float32),
        jax.ShapeDtypeStruct((N, H), jnp.float32),
    ],
    scratch_shapes=[pltpu.VMEM((P, HID), jnp.float32)],
)


def kernel(features, edge_index, simlar, W_fc, b_fc, W1, b1, W2, Wp, bp):
    x = features[0]
    h0, alpha_t = _fc(x, W_fc, b_fc.reshape(1, H), simlar.reshape(N, 1))
    edge_r = edge_index.reshape(NC, 2, 2, NS, NCH, CW)
    ones = jnp.ones((CW, 16), jnp.float32)
    z64 = jnp.zeros((RC, H), jnp.float32)
    z16 = jnp.zeros((RC, 16), jnp.float32)
    z = _sc_call(edge_r, h0, alpha_t, ones, z64, z16)
    a, h = _att(z, W1, b1.reshape(1, HID), W2, Wp, bp.reshape(1, OUT))
    return (a, h)


# speedup vs baseline: 2.4314x; 1.0142x over previous
import functools

import jax
import jax.numpy as jnp
from jax import lax
from jax.experimental import pallas as pl
from jax.experimental.pallas import tpu as pltpu
from jax.experimental.pallas import tpu_sc as plsc

N = 10000
P = 4
E = 160000
D = 256
H = 64
HID = 128
OUT = 3
ALPHA = 0.15

NC = 2
NS = 16
EW = E // NS
CW = 80
NCH = EW // CW
RC = 80
NRC = N // RC
MAXJ = (NRC + NS - 1) // NS
NB = 10
BN = N // NB


def _fc_body(x_ref, w_ref, b_ref, s_ref, h0_ref, al_ref):
    h = jnp.dot(x_ref[...], w_ref[...], preferred_element_type=jnp.float32)
    h = jnp.maximum(h + b_ref[...], 0.0)
    h0_ref[...] = h
    al_ref[...] = ALPHA * s_ref[...] * h


_fc = pl.pallas_call(
    _fc_body,
    grid=(NB,),
    in_specs=[
        pl.BlockSpec((BN, D), lambda i: (i, 0)),
        pl.BlockSpec((D, H), lambda i: (0, 0)),
        pl.BlockSpec((1, H), lambda i: (0, 0)),
        pl.BlockSpec((BN, 1), lambda i: (i, 0)),
    ],
    out_specs=[
        pl.BlockSpec((BN, H), lambda i: (i, 0)),
        pl.BlockSpec((BN, H), lambda i: (i, 0)),
    ],
    out_shape=[
        jax.ShapeDtypeStruct((N, H), jnp.float32),
        jax.ShapeDtypeStruct((N + RC, H), jnp.float32),
    ],
)


NBUF = 7
NGRP = NCH // NBUF
NTAIL = NCH - NGRP * NBUF


def _edge_pass(table_ref, acc_g, deg_g, src_v, dst_v, msg_v, ones_v,
               gsem, ssem, dsem, *, with_deg):
    for t in range(NBUF):
        pltpu.async_copy(table_ref.at[src_v.at[t]], msg_v.at[t], gsem.at[t])

    def _grp(j2, _):
        j0 = j2 * NBUF
        for t in range(NBUF):
            j = j0 + t
            pltpu.make_async_copy(
                table_ref.at[src_v.at[t]], msg_v.at[t], gsem.at[t]).wait()
            if with_deg:
                pltpu.async_copy(ones_v, deg_g.at[dst_v.at[j]], dsem, add=True)
            pltpu.async_copy(msg_v.at[t], acc_g.at[dst_v.at[j]], ssem.at[t],
                             add=True)
        for t in range(NBUF):
            j = j0 + t
            pltpu.make_async_copy(
                msg_v.at[t], acc_g.at[dst_v.at[j]], ssem.at[t]).wait()
            nxt = jnp.minimum(j + NBUF, NCH - 1)
            pltpu.async_copy(table_ref.at[src_v.at[nxt]], msg_v.at[t],
                             gsem.at[t])
        return _
    lax.fori_loop(0, NGRP, _grp, None)

    j0 = NGRP * NBUF
    for t in range(NTAIL):
        j = j0 + t
        pltpu.make_async_copy(
            table_ref.at[src_v.at[0]], msg_v.at[t], gsem.at[t]).wait()
        if with_deg:
            pltpu.async_copy(ones_v, deg_g.at[dst_v.at[j]], dsem, add=True)
        pltpu.async_copy(msg_v.at[t], acc_g.at[dst_v.at[j]], ssem.at[t],
                         add=True)
    for t in range(NTAIL):
        j = j0 + t
        pltpu.make_async_copy(
            msg_v.at[t], acc_g.at[dst_v.at[j]], ssem.at[t]).wait()
    for t in range(NTAIL, NBUF):
        pltpu.make_async_copy(
            table_ref.at[src_v.at[0]], msg_v.at[t], gsem.at[t]).wait()
    if with_deg:
        def _dwait(j, _):
            pltpu.make_async_copy(ones_v, deg_g.at[dst_v.at[0]], dsem).wait()
            return _
        lax.fori_loop(0, NCH, _dwait, None)


def _combine(p, s, z_ref, alpha_ref, z64_ref, z16_ref, acc_g, deg_g,
             deg_c, acc_c, al_c, csem, *, first):
    def _base(k):
        return jnp.minimum(s + k * NS, NRC) * RC

    def _deg_cp(k, b):
        return pltpu.make_async_copy(
            deg_g.at[pl.ds(_base(k), RC), :], deg_c.at[b], csem.at[0, b])

    def _acc_cp(k, b):
        return pltpu.make_async_copy(
            acc_g.at[pl.ds(_base(k), RC), :], acc_c.at[b], csem.at[1, b])

    def _al_cp(k, b):
        return pltpu.make_async_copy(
            alpha_ref.at[pl.ds(_base(k), RC), :], al_c.at[b], csem.at[2, b])

    def _zst_cp(k, b):
        return pltpu.make_async_copy(
            acc_c.at[b], z_ref.at[p, pl.ds(_base(k), RC), :], csem.at[3, b])

    def _rz_cp(k, b):
        return pltpu.make_async_copy(
            z64_ref, acc_g.at[pl.ds(_base(k), RC), :], csem.at[4, b])

    def _rz16_cp(k, b):
        return pltpu.make_async_copy(
            z16_ref, deg_g.at[pl.ds(_base(k), RC), :], csem.at[5, b])

    def _issue_loads(k, b):
        _deg_cp(k, b).start()
        _acc_cp(k, b).start()
        _al_cp(k, b).start()

    def _wait_loads(k, b):
        _deg_cp(k, b).wait()
        _acc_cp(k, b).wait()
        _al_cp(k, b).wait()

    def _issue_stores(k, b):
        _zst_cp(k, b).start()
        _rz_cp(k, b).start()
        if not first:
            _rz16_cp(k, b).start()

    def _wait_stores(k, b):
        _zst_cp(k, b).wait()
        _rz_cp(k, b).wait()
        if not first:
            _rz16_cp(k, b).wait()

    _issue_loads(0, 0)
    for k in range(MAXJ):
        b = k % 2
        _wait_loads(k, b)

        def _rows(r, _2, b=b):
            rd = (1.0 - ALPHA) / jnp.maximum(deg_c[b, r, :], 1.0)
            for u in range(4):
                sl = pl.ds(u * 16, 16)
                acc_c[b, r, sl] = acc_c[b, r, sl] * rd + al_c[b, r, sl]
            return _2
        lax.fori_loop(0, RC, _rows, None)
        _issue_stores(k, b)
        if k + 1 < MAXJ:
            if k >= 1:
                _wait_stores(k - 1, 1 - b)
            _issue_loads(k + 1, 1 - b)
    _wait_stores(MAXJ - 1, (MAXJ - 1) % 2)


def _sc_body(edge_ref, h0_ref, alpha_ref, ones_ref, z64_ref, z16_ref, z_ref,
             acc_sh, deg_sh, src_v, dst_v, msg_v, ones_v,
             deg_c, acc_c, al_c, gsem, ssem, dsem, csem):
    c = lax.axis_index("c")
    s = lax.axis_index("s")

    pltpu.sync_copy(ones_ref, ones_v)

    def _zero(k, _):
        base = jnp.minimum(s + k * NS, NRC) * RC
        pltpu.sync_copy(z64_ref, acc_sh.at[pl.ds(base, RC), :])
        pltpu.sync_copy(z16_ref, deg_sh.at[pl.ds(base, RC), :])
        return _
    lax.fori_loop(0, MAXJ, _zero, None)

    for g in range(2):
        p = 2 * c + g
        acc_g = acc_sh
        deg_g = deg_sh

        pltpu.sync_copy(edge_ref.at[c, g, 0, s], src_v)
        pltpu.sync_copy(edge_ref.at[c, g, 1, s], dst_v)
        plsc.subcore_barrier()

        _edge_pass(h0_ref, acc_g, deg_g, src_v, dst_v, msg_v, ones_v,
                   gsem, ssem, dsem, with_deg=True)
        plsc.subcore_barrier()

        _combine(p, s, z_ref, alpha_ref, z64_ref, z16_ref, acc_g, deg_g,
                 deg_c, acc_c, al_c, csem, first=True)
        plsc.subcore_barrier()

        _edge_pass(z_ref.at[p], acc_g, deg_g, src_v, dst_v, msg_v, ones_v,
                   gsem, ssem, dsem, with_deg=False)
        plsc.subcore_barrier()

        _combine(p, s, z_ref, alpha_ref, z64_ref, z16_ref, acc_g, deg_g,
                 deg_c, acc_c, al_c, csem, first=False)


_sc_call = pl.kernel(
    _sc_body,
    out_type=jax.ShapeDtypeStruct((P, N + RC, H), jnp.float32),
    mesh=plsc.VectorSubcoreMesh(
        core_axis_name="c", subcore_axis_name="s",
        num_cores=NC, num_subcores=NS),
    compiler_params=pltpu.CompilerParams(use_tc_tiling_on_sc=False),
    scratch_types=[
        pltpu.VMEM_SHARED((N + RC, H), jnp.float32),
        pltpu.VMEM_SHARED((N + RC, 16), jnp.float32),
        pltpu.VMEM((NCH, CW), jnp.int32),
        pltpu.VMEM((NCH, CW), jnp.int32),
        pltpu.VMEM((NBUF, CW, H), jnp.float32),
        pltpu.VMEM((CW, 16), jnp.float32),
        pltpu.VMEM((2, RC, 16), jnp.float32),
        pltpu.VMEM((2, RC, H), jnp.float32),
        pltpu.VMEM((2, RC, H), jnp.float32),
        pltpu.SemaphoreType.DMA((NBUF,)),
        pltpu.SemaphoreType.DMA((NBUF,)),
        pltpu.SemaphoreType.DMA,
        pltpu.SemaphoreType.DMA((6, 2)),
    ],
)


def _att_body(z_ref, w1_ref, b1_ref, w2_ref, wp_ref, bp_ref, a_ref, h_ref, sg):
    ph = pl.program_id(0)
    i = pl.program_id(1)

    @pl.when(ph == 0)
    def _phase0():
        @pl.when(i == 0)
        def _init():
            sg[...] = jnp.zeros((P, HID), jnp.float32)
        zb = z_ref[...]
        parts = []
        for q in range(P):
            tq = jnp.dot(zb[q], w1_ref[...], preferred_element_type=jnp.float32)
            tq = jnp.tanh(tq + b1_ref[...])
            parts.append(jnp.sum(tq, axis=0))
        sg[...] = sg[...] + jnp.stack(parts)

    @pl.when(ph == 1)
    def _phase1():
        S = sg[...]
        w = jnp.dot(S, w2_ref[...], preferred_element_type=jnp.float32) / N
        w = w - jnp.max(w, axis=0, keepdims=True)
        ew = jnp.exp(w)
        beta = ew / jnp.sum(ew, axis=0, keepdims=True)
        zb = z_ref[...]
        hacc = beta[0, 0] * zb[0]
        for q in range(1, P):
            hacc = hacc + beta[q, 0] * zb[q]
        h_ref[...] = hacc
        a_ref[...] = jnp.dot(
            hacc, wp_ref[...], preferred_element_type=jnp.float32) + bp_ref[...]


_att = pl.pallas_call(
    _att_body,
    grid=(2, NB),
    in_specs=[
        pl.BlockSpec((P, BN, H), lambda ph, i: (0, i, 0)),
        pl.BlockSpec((H, HID), lambda ph, i: (0, 0)),
        pl.BlockSpec((1, HID), lambda ph, i: (0, 0)),
        pl.BlockSpec((HID, 1), lambda ph, i: (0, 0)),
        pl.BlockSpec((H, OUT), lambda ph, i: (0, 0)),
        pl.BlockSpec((1, OUT), lambda ph, i: (0, 0)),
    ],
    out_specs=[
        pl.BlockSpec((BN, OUT), lambda ph, i: (i, 0)),
        pl.BlockSpec((BN, H), lambda ph, i: (i, 0)),
    ],
    out_shape=[
        jax.ShapeDtypeStruct((N, OUT), jnp.float32),
        jax.ShapeDtypeStruct((N, H), jnp.float32),
    ],
    scratch_shapes=[pltpu.VMEM((P, HID), jnp.float32)],
)


def kernel(features, edge_index, simlar, W_fc, b_fc, W1, b1, W2, Wp, bp):
    x = features[0]
    h0, alpha_t = _fc(x, W_fc, b_fc.reshape(1, H), simlar.reshape(N, 1))
    edge_r = edge_index.reshape(NC, 2, 2, NS, NCH, CW)
    ones = jnp.ones((CW, 16), jnp.float32)
    z64 = jnp.zeros((RC, H), jnp.float32)
    z16 = jnp.zeros((RC, 16), jnp.float32)
    z = _sc_call(edge_r, h0, alpha_t, ones, z64, z16)
    a, h = _att(z, W1, b1.reshape(1, HID), W2, Wp, bp.reshape(1, OUT))
    return (a, h)


# speedup vs baseline: 2.4349x; 1.0014x over previous
import jax
import jax.numpy as jnp
from jax import lax
from jax.experimental import pallas as pl
from jax.experimental.pallas import tpu as pltpu
from jax.experimental.pallas import tpu_sc as plsc

N = 10000
P = 4
E = 160000
D = 256
H = 64
HID = 128
OUT = 3
ALPHA = 0.15

NC = 2
NS = 16
EW = E // NS
CW = 80
NCH = EW // CW
RC = 80
NRC = N // RC
MAXJ = (NRC + NS - 1) // NS
NB = 10
BN = N // NB


def _fc_body(x_ref, w_ref, b_ref, s_ref, h0_ref, al_ref):
    h = jnp.dot(x_ref[...], w_ref[...], preferred_element_type=jnp.float32)
    h = jnp.maximum(h + b_ref[...], 0.0)
    h0_ref[...] = h
    al_ref[...] = ALPHA * s_ref[...] * h


_fc = pl.pallas_call(
    _fc_body,
    grid=(NB,),
    in_specs=[
        pl.BlockSpec((BN, D), lambda i: (i, 0)),
        pl.BlockSpec((D, H), lambda i: (0, 0)),
        pl.BlockSpec((1, H), lambda i: (0, 0)),
        pl.BlockSpec((BN, 1), lambda i: (i, 0)),
    ],
    out_specs=[
        pl.BlockSpec((BN, H), lambda i: (i, 0)),
        pl.BlockSpec((BN, H), lambda i: (i, 0)),
    ],
    out_shape=[
        jax.ShapeDtypeStruct((N, H), jnp.float32),
        jax.ShapeDtypeStruct((N + RC, H), jnp.float32),
    ],
)


NBUF = 7
NGRP = NCH // NBUF
NTAIL = NCH - NGRP * NBUF


def _edge_pass(table_ref, acc_g, deg_g, src_v, dst_v, msg_v, ones_v,
               gsem, ssem, dsem, *, with_deg):
    for t in range(NBUF):
        pltpu.async_copy(table_ref.at[src_v.at[t]], msg_v.at[t], gsem.at[t])

    def _grp(j2, _):
        j0 = j2 * NBUF
        for t in range(NBUF):
            j = j0 + t
            pltpu.make_async_copy(
                table_ref.at[src_v.at[t]], msg_v.at[t], gsem.at[t]).wait()
            if with_deg:
                pltpu.async_copy(ones_v, deg_g.at[dst_v.at[j]], dsem, add=True)
            pltpu.async_copy(msg_v.at[t], acc_g.at[dst_v.at[j]], ssem.at[t],
                             add=True)
        for t in range(NBUF):
            j = j0 + t
            pltpu.make_async_copy(
                msg_v.at[t], acc_g.at[dst_v.at[j]], ssem.at[t]).wait()
            nxt = jnp.minimum(j + NBUF, NCH - 1)
            pltpu.async_copy(table_ref.at[src_v.at[nxt]], msg_v.at[t],
                             gsem.at[t])
        return _
    lax.fori_loop(0, NGRP, _grp, None)

    j0 = NGRP * NBUF
    for t in range(NTAIL):
        j = j0 + t
        pltpu.make_async_copy(
            table_ref.at[src_v.at[0]], msg_v.at[t], gsem.at[t]).wait()
        if with_deg:
            pltpu.async_copy(ones_v, deg_g.at[dst_v.at[j]], dsem, add=True)
        pltpu.async_copy(msg_v.at[t], acc_g.at[dst_v.at[j]], ssem.at[t],
                         add=True)
    for t in range(NTAIL):
        j = j0 + t
        pltpu.make_async_copy(
            msg_v.at[t], acc_g.at[dst_v.at[j]], ssem.at[t]).wait()
    for t in range(NTAIL, NBUF):
        pltpu.make_async_copy(
            table_ref.at[src_v.at[0]], msg_v.at[t], gsem.at[t]).wait()
    if with_deg:
        def _dwait(j, _):
            pltpu.make_async_copy(ones_v, deg_g.at[dst_v.at[0]], dsem).wait()
            return _
        lax.fori_loop(0, NCH, _dwait, None)


def _combine(p, s, z_ref, alpha_ref, z64_ref, z16_ref, acc_g, deg_g,
             deg_c, acc_c, al_c, csem, *, first):
    def _base(k):
        return jnp.minimum(s + k * NS, NRC) * RC

    def _deg_cp(k, b):
        return pltpu.make_async_copy(
            deg_g.at[pl.ds(_base(k), RC), :], deg_c.at[b], csem.at[0, b])

    def _acc_cp(k, b):
        return pltpu.make_async_copy(
            acc_g.at[pl.ds(_base(k), RC), :], acc_c.at[b], csem.at[1, b])

    def _al_cp(k, b):
        return pltpu.make_async_copy(
            alpha_ref.at[pl.ds(_base(k), RC), :], al_c.at[b], csem.at[2, b])

    def _zst_cp(k, b):
        return pltpu.make_async_copy(
            acc_c.at[b], z_ref.at[p, pl.ds(_base(k), RC), :], csem.at[3, b])

    def _rz_cp(k, b):
        return pltpu.make_async_copy(
            z64_ref, acc_g.at[pl.ds(_base(k), RC), :], csem.at[4, b])

    def _rz16_cp(k, b):
        return pltpu.make_async_copy(
            z16_ref, deg_g.at[pl.ds(_base(k), RC), :], csem.at[5, b])

    def _issue_loads(k, b):
        _deg_cp(k, b).start()
        _acc_cp(k, b).start()
        _al_cp(k, b).start()

    def _wait_loads(k, b):
        _deg_cp(k, b).wait()
        _acc_cp(k, b).wait()
        _al_cp(k, b).wait()

    def _issue_stores(k, b):
        _zst_cp(k, b).start()
        _rz_cp(k, b).start()
        if not first:
            _rz16_cp(k, b).start()

    def _wait_stores(k, b):
        _zst_cp(k, b).wait()
        _rz_cp(k, b).wait()
        if not first:
            _rz16_cp(k, b).wait()

    _issue_loads(0, 0)
    for k in range(MAXJ):
        b = k % 2
        _wait_loads(k, b)

        def _rows(r, _2, b=b):
            rd = (1.0 - ALPHA) / jnp.maximum(deg_c[b, r, :], 1.0)
            for u in range(4):
                sl = pl.ds(u * 16, 16)
                acc_c[b, r, sl] = acc_c[b, r, sl] * rd + al_c[b, r, sl]
            return _2
        lax.fori_loop(0, RC, _rows, None)
        _issue_stores(k, b)
        if k + 1 < MAXJ:
            if k >= 1:
                _wait_stores(k - 1, 1 - b)
            _issue_loads(k + 1, 1 - b)
    _wait_stores(MAXJ - 1, (MAXJ - 1) % 2)


def _sc_body(edge_ref, h0_ref, alpha_ref, ones_ref, z64_ref, z16_ref, z_ref,
             acc_sh, deg_sh, src_v, dst_v, msg_v, ones_v,
             deg_c, acc_c, al_c, gsem, ssem, dsem, csem):
    c = lax.axis_index("c")
    s = lax.axis_index("s")

    pltpu.sync_copy(ones_ref, ones_v)

    def _zero(k, _):
        base = jnp.minimum(s + k * NS, NRC) * RC
        pltpu.sync_copy(z64_ref, acc_sh.at[pl.ds(base, RC), :])
        pltpu.sync_copy(z16_ref, deg_sh.at[pl.ds(base, RC), :])
        return _
    lax.fori_loop(0, MAXJ, _zero, None)

    for g in range(2):
        p = 2 * c + g
        acc_g = acc_sh
        deg_g = deg_sh

        pltpu.sync_copy(edge_ref.at[c, g, 0, s], src_v)
        pltpu.sync_copy(edge_ref.at[c, g, 1, s], dst_v)
        plsc.subcore_barrier()

        _edge_pass(h0_ref, acc_g, deg_g, src_v, dst_v, msg_v, ones_v,
                   gsem, ssem, dsem, with_deg=True)
        plsc.subcore_barrier()

        _combine(p, s, z_ref, alpha_ref, z64_ref, z16_ref, acc_g, deg_g,
                 deg_c, acc_c, al_c, csem, first=True)
        plsc.subcore_barrier()

        _edge_pass(z_ref.at[p], acc_g, deg_g, src_v, dst_v, msg_v, ones_v,
                   gsem, ssem, dsem, with_deg=False)
        plsc.subcore_barrier()

        _combine(p, s, z_ref, alpha_ref, z64_ref, z16_ref, acc_g, deg_g,
                 deg_c, acc_c, al_c, csem, first=False)


_sc_call = pl.kernel(
    _sc_body,
    out_type=jax.ShapeDtypeStruct((P, N + RC, H), jnp.float32),
    mesh=plsc.VectorSubcoreMesh(
        core_axis_name="c", subcore_axis_name="s",
        num_cores=NC, num_subcores=NS),
    compiler_params=pltpu.CompilerParams(use_tc_tiling_on_sc=False),
    scratch_types=[
        pltpu.VMEM_SHARED((N + RC, H), jnp.float32),
        pltpu.VMEM_SHARED((N + RC, 16), jnp.float32),
        pltpu.VMEM((NCH, CW), jnp.int32),
        pltpu.VMEM((NCH, CW), jnp.int32),
        pltpu.VMEM((NBUF, CW, H), jnp.float32),
        pltpu.VMEM((CW, 16), jnp.float32),
        pltpu.VMEM((2, RC, 16), jnp.float32),
        pltpu.VMEM((2, RC, H), jnp.float32),
        pltpu.VMEM((2, RC, H), jnp.float32),
        pltpu.SemaphoreType.DMA((NBUF,)),
        pltpu.SemaphoreType.DMA((NBUF,)),
        pltpu.SemaphoreType.DMA,
        pltpu.SemaphoreType.DMA((6, 2)),
    ],
)


def _att_body(z_ref, w1_ref, b1_ref, w2_ref, wp_ref, bp_ref, a_ref, h_ref, sg):
    ph = pl.program_id(0)
    i = pl.program_id(1)

    @pl.when(ph == 0)
    def _phase0():
        @pl.when(i == 0)
        def _init():
            sg[...] = jnp.zeros((P, HID), jnp.float32)
        zb = z_ref[...]
        parts = []
        for q in range(P):
            tq = jnp.dot(zb[q], w1_ref[...], preferred_element_type=jnp.float32)
            tq = jnp.tanh(tq + b1_ref[...])
            parts.append(jnp.sum(tq, axis=0))
        sg[...] = sg[...] + jnp.stack(parts)

    @pl.when(ph == 1)
    def _phase1():
        S = sg[...]
        w = jnp.dot(S, w2_ref[...], preferred_element_type=jnp.float32) / N
        w = w - jnp.max(w, axis=0, keepdims=True)
        ew = jnp.exp(w)
        beta = ew / jnp.sum(ew, axis=0, keepdims=True)
        zb = z_ref[...]
        hacc = beta[0, 0] * zb[0]
        for q in range(1, P):
            hacc = hacc + beta[q, 0] * zb[q]
        h_ref[...] = hacc
        a_ref[...] = jnp.dot(
            hacc, wp_ref[...], preferred_element_type=jnp.float32) + bp_ref[...]


_att = pl.pallas_call(
    _att_body,
    grid=(2, NB),
    in_specs=[
        pl.BlockSpec((P, BN, H), lambda ph, i: (0, i, 0)),
        pl.BlockSpec((H, HID), lambda ph, i: (0, 0)),
        pl.BlockSpec((1, HID), lambda ph, i: (0, 0)),
        pl.BlockSpec((HID, 1), lambda ph, i: (0, 0)),
        pl.BlockSpec((H, OUT), lambda ph, i: (0, 0)),
        pl.BlockSpec((1, OUT), lambda ph, i: (0, 0)),
    ],
    out_specs=[
        pl.BlockSpec((BN, OUT), lambda ph, i: (i, 0)),
        pl.BlockSpec((BN, H), lambda ph, i: (i, 0)),
    ],
    out_shape=[
        jax.ShapeDtypeStruct((N, OUT), jnp.float32),
        jax.ShapeDtypeStruct((N, H), jnp.float32),
    ],
    scratch_shapes=[pltpu.VMEM((P, HID), jnp.float32)],
)


def kernel(features, edge_index, simlar, W_fc, b_fc, W1, b1, W2, Wp, bp):
    x = features[0]
    h0, alpha_t = _fc(x, W_fc, b_fc.reshape(1, H), simlar.reshape(N, 1))
    edge_r = edge_index.reshape(NC, 2, 2, NS, NCH, CW)
    ones = jnp.ones((CW, 16), jnp.float32)
    z64 = jnp.zeros((RC, H), jnp.float32)
    z16 = jnp.zeros((RC, 16), jnp.float32)
    z = _sc_call(edge_r, h0, alpha_t, ones, z64, z16)
    a, h = _att(z, W1, b1.reshape(1, HID), W2, Wp, bp.reshape(1, OUT))
    return (a, h)
